# baseline (speedup 1.0000x reference)
import jax
import jax.numpy as jnp
from jax import lax
from jax.experimental import pallas as pl
from jax.experimental.pallas import tpu as pltpu
from jax.experimental.pallas import tpu_sc as plsc

_NC = 2
_NS = 16
_NW = _NC * _NS
_CHUNK = 800
_SPLITS = 4
_RE_LINES = 80


def _repack(W):
    V, dim = W.shape
    lines = V // 4
    n_chunks = lines // _RE_LINES
    mesh = plsc.VectorSubcoreMesh(core_axis_name="c", subcore_axis_name="s")

    @pl.kernel(
        out_type=jax.ShapeDtypeStruct((lines, 128), W.dtype),
        mesh=mesh,
        scratch_types=[
            pltpu.VMEM((4 * _RE_LINES, dim), jnp.float32),
            pltpu.VMEM((_RE_LINES, 128), jnp.float32),
        ],
    )
    def repack_kernel(w_hbm, o_hbm, vin_v, vout_v):
        wid = lax.axis_index("s") * _NC + lax.axis_index("c")
        iters = n_chunks // _NW + 1

        @pl.loop(0, iters)
        def _(i):
            cid = i * _NW + wid

            @pl.when(cid < n_chunks)
            def _():
                pltpu.sync_copy(w_hbm.at[pl.ds(cid * 4 * _RE_LINES, 4 * _RE_LINES)], vin_v)

                @plsc.parallel_loop(0, _RE_LINES)
                def _(m):
                    for s in range(8):
                        vout_v.at[m, pl.ds(s * 16, 16)][...] = (
                            vin_v.at[4 * m + s // 2, pl.ds((s % 2) * 16, 16)][...]
                        )

                pltpu.sync_copy(vout_v, o_hbm.at[pl.ds(cid * _RE_LINES, _RE_LINES)])

    return repack_kernel(W)


def _gather_part(W, idx_part):
    n = idx_part.shape[0]
    b_per_w = n // _NW
    n_chunks = b_per_w // _CHUNK
    mesh = plsc.VectorSubcoreMesh(core_axis_name="c", subcore_axis_name="s")

    @pl.kernel(
        out_type=jax.ShapeDtypeStruct((n, W.shape[1]), W.dtype),
        mesh=mesh,
        compiler_params=pltpu.CompilerParams(use_tc_tiling_on_sc=False),
        scratch_types=[
            pltpu.VMEM((_CHUNK,), jnp.int32),
            pltpu.VMEM((_CHUNK, W.shape[1]), jnp.float32),
            pltpu.SemaphoreType.DMA,
        ],
    )
    def gather_kernel(w_hbm, i_hbm, o_hbm, idx_v, rows_v, sem):
        wid = lax.axis_index("s") * _NC + lax.axis_index("c")
        base = wid * b_per_w

        @pl.loop(0, n_chunks)
        def _(j):
            off = base + j * _CHUNK
            pltpu.sync_copy(i_hbm.at[pl.ds(off, _CHUNK)], idx_v)
            pltpu.async_copy(w_hbm.at[idx_v], rows_v, sem).wait()
            pltpu.sync_copy(rows_v, o_hbm.at[pl.ds(off, _CHUNK)])

    return gather_kernel(W, idx_part)


def kernel(token_ids, W):
    B, L = token_ids.shape
    n = B * L
    dim = W.shape[1]
    idx = token_ids.reshape(n)

    w2 = _repack(W).reshape(W.shape)

    part = n // _SPLITS
    outs = [
        _gather_part(w2, idx[p * part:(p + 1) * part]).reshape(B // _SPLITS, L, dim)
        for p in range(_SPLITS)
    ]
    return jnp.concatenate(outs, axis=0)


# speedup vs baseline: 1.1088x; 1.1088x over previous
import jax
import jax.numpy as jnp
from jax import lax
from jax.experimental import pallas as pl
from jax.experimental.pallas import tpu as pltpu
from jax.experimental.pallas import tpu_sc as plsc

_NC = 2
_NS = 16
_NW = _NC * _NS
_CHUNK = 800
_SPLITS = 8
_RE_LINES = 80


def _repack(W):
    V, dim = W.shape
    lines = V // 4
    n_chunks = lines // _RE_LINES
    mesh = plsc.VectorSubcoreMesh(core_axis_name="c", subcore_axis_name="s")

    @pl.kernel(
        out_type=jax.ShapeDtypeStruct((lines, 128), W.dtype),
        mesh=mesh,
        scratch_types=[
            pltpu.VMEM((4 * _RE_LINES, dim), jnp.float32),
            pltpu.VMEM((_RE_LINES, 128), jnp.float32),
        ],
    )
    def repack_kernel(w_hbm, o_hbm, vin_v, vout_v):
        wid = lax.axis_index("s") * _NC + lax.axis_index("c")
        iters = n_chunks // _NW + 1

        @pl.loop(0, iters)
        def _(i):
            cid = i * _NW + wid

            @pl.when(cid < n_chunks)
            def _():
                pltpu.sync_copy(w_hbm.at[pl.ds(cid * 4 * _RE_LINES, 4 * _RE_LINES)], vin_v)

                @plsc.parallel_loop(0, _RE_LINES)
                def _(m):
                    for s in range(8):
                        vout_v.at[m, pl.ds(s * 16, 16)][...] = (
                            vin_v.at[4 * m + s // 2, pl.ds((s % 2) * 16, 16)][...]
                        )

                pltpu.sync_copy(vout_v, o_hbm.at[pl.ds(cid * _RE_LINES, _RE_LINES)])

    return repack_kernel(W)


def _gather_part(W, idx_part):
    n = idx_part.shape[0]
    b_per_w = n // _NW
    n_chunks = b_per_w // _CHUNK
    mesh = plsc.VectorSubcoreMesh(core_axis_name="c", subcore_axis_name="s")

    @pl.kernel(
        out_type=jax.ShapeDtypeStruct((n, W.shape[1]), W.dtype),
        mesh=mesh,
        compiler_params=pltpu.CompilerParams(use_tc_tiling_on_sc=False),
        scratch_types=[
            pltpu.VMEM((_CHUNK,), jnp.int32),
            pltpu.VMEM((_CHUNK, W.shape[1]), jnp.float32),
            pltpu.SemaphoreType.DMA,
        ],
    )
    def gather_kernel(w_hbm, i_hbm, o_hbm, idx_v, rows_v, sem):
        wid = lax.axis_index("s") * _NC + lax.axis_index("c")
        base = wid * b_per_w

        @pl.loop(0, n_chunks)
        def _(j):
            off = base + j * _CHUNK
            pltpu.sync_copy(i_hbm.at[pl.ds(off, _CHUNK)], idx_v)
            pltpu.async_copy(w_hbm.at[idx_v], rows_v, sem).wait()
            pltpu.sync_copy(rows_v, o_hbm.at[pl.ds(off, _CHUNK)])

    return gather_kernel(W, idx_part)


def kernel(token_ids, W):
    B, L = token_ids.shape
    n = B * L
    dim = W.shape[1]
    idx = token_ids.reshape(n)

    part = n // _SPLITS
    outs = [
        _gather_part(W, idx[p * part:(p + 1) * part]).reshape(B // _SPLITS, L, dim)
        for p in range(_SPLITS)
    ]
    return jnp.concatenate(outs, axis=0)


# speedup vs baseline: 1.1172x; 1.0076x over previous
import jax
import jax.numpy as jnp
from jax import lax
from jax.experimental import pallas as pl
from jax.experimental.pallas import tpu as pltpu
from jax.experimental.pallas import tpu_sc as plsc

_NC = 2
_NS = 16
_NW = _NC * _NS
_CHUNK = 800
_SPLITS = 4


def _gather_part(W, idx_part):
    n = idx_part.shape[0]
    b_per_w = n // _NW
    n_chunks = b_per_w // _CHUNK
    mesh = plsc.VectorSubcoreMesh(core_axis_name="c", subcore_axis_name="s")

    @pl.kernel(
        out_type=jax.ShapeDtypeStruct((n, W.shape[1]), W.dtype),
        mesh=mesh,
        compiler_params=pltpu.CompilerParams(use_tc_tiling_on_sc=False),
        scratch_types=[
            pltpu.VMEM((_CHUNK,), jnp.int32),
            pltpu.VMEM((_CHUNK, W.shape[1]), jnp.float32),
            pltpu.SemaphoreType.DMA,
        ],
    )
    def gather_kernel(w_hbm, i_hbm, o_hbm, idx_v, rows_v, sem):
        wid = lax.axis_index("s") * _NC + lax.axis_index("c")
        base = wid * b_per_w

        @pl.loop(0, n_chunks)
        def _(j):
            off = base + j * _CHUNK
            pltpu.sync_copy(i_hbm.at[pl.ds(off, _CHUNK)], idx_v)
            pltpu.async_copy(w_hbm.at[idx_v], rows_v, sem).wait()
            pltpu.sync_copy(rows_v, o_hbm.at[pl.ds(off, _CHUNK)])

    return gather_kernel(W, idx_part)


def kernel(token_ids, W):
    B, L = token_ids.shape
    n = B * L
    dim = W.shape[1]
    idx = token_ids.reshape(n)

    part = n // _SPLITS
    outs = [
        _gather_part(W, idx[p * part:(p + 1) * part]).reshape(B // _SPLITS, L, dim)
        for p in range(_SPLITS)
    ]
    return jnp.concatenate(outs, axis=0)
